# baseline (speedup 1.0000x reference)
import functools

import jax
import jax.numpy as jnp
from jax import lax
from jax.experimental import pallas as pl
from jax.experimental.pallas import tpu as pltpu
from jax.experimental.pallas import tpu_sc as plsc

N_NODES = 10000
N_EDGES = 320000
D_IN = 128
D_OUT = 128

NC = 2
NS = 16
NW = NC * NS
L = 16

K = 128
G = K // L
E_T = N_EDGES // NW
NBF = E_T // K
TAIL = E_T - NBF * K
N_PAD = 10240
RPT = N_PAD // NS


def _sc_aggregate(feature, src, dst, w):
    mesh = plsc.VectorSubcoreMesh(core_axis_name="c", subcore_axis_name="s")

    @functools.partial(
        pl.kernel,
        out_type=(
            jax.ShapeDtypeStruct((NC, N_PAD, D_IN), jnp.float32),
            jax.ShapeDtypeStruct((NC, NS, N_PAD), jnp.float32),
        ),
        mesh=mesh,
        compiler_params=pltpu.CompilerParams(needs_layout_passes=False),
        scratch_types=[
            pltpu.VMEM((K,), jnp.int32),
            pltpu.VMEM((K,), jnp.int32),
            pltpu.VMEM((K,), jnp.int32),
            pltpu.VMEM((K,), jnp.int32),
            pltpu.VMEM((K,), jnp.float32),
            pltpu.VMEM((K,), jnp.float32),
            pltpu.VMEM((K,), jnp.int32),
            pltpu.VMEM((K,), jnp.int32),
            pltpu.VMEM((TAIL,), jnp.int32),
            pltpu.VMEM((TAIL,), jnp.int32),
            pltpu.VMEM((TAIL,), jnp.float32),
            pltpu.VMEM((K, D_IN), jnp.float32),
            pltpu.VMEM((K, D_IN), jnp.float32),
            pltpu.VMEM((N_PAD,), jnp.float32),
            pltpu.VMEM_SHARED((N_PAD, D_IN), jnp.float32),
            pltpu.SemaphoreType.DMA,
            pltpu.SemaphoreType.DMA,
            pltpu.SemaphoreType.DMA,
            pltpu.SemaphoreType.DMA,
        ],
    )
    def agg(feat_hbm, src_hbm, dst_hbm, w_hbm, out_hbm, cnt_hbm,
            src_a, src_b, dst_a, dst_b, w_a, w_b, dsc_a, dsc_b,
            src_t, dst_t, w_t,
            rows_a, rows_b, cnt_v, acc_sh,
            gsem_a, gsem_b, isem_a, isem_b):
        c = lax.axis_index("c")
        s = lax.axis_index("s")
        base = (c * NS + s) * E_T

        zvec = jnp.zeros((L,), jnp.float32)

        def zrow(r, carry):
            for q in range(D_IN // L):
                rows_a[r, pl.ds(q * L, L)] = zvec
            return carry

        lax.fori_loop(0, K, zrow, 0)

        def zcnt(r, carry):
            cnt_v[pl.ds(r * L, L)] = zvec
            return carry

        lax.fori_loop(0, N_PAD // L, zcnt, 0)

        for r in range(RPT // K):
            pltpu.sync_copy(rows_a,
                            acc_sh.at[pl.ds(s * RPT + r * K, K)])

        plsc.subcore_barrier()

        ones = jnp.full((L,), 1.0, jnp.float32)

        def mul_block(dst_v, w_v, dsc_v, rows_v, n_groups):
            def group(g, carry):
                dvec = dst_v[pl.ds(g * L, L)]
                plsc.addupdate_scatter(cnt_v, [dvec], ones)
                dsc_v[pl.ds(g * L, L)] = dvec
                wg = w_v[pl.ds(g * L, L)]
                for t in range(L):
                    wb = wg.at[jnp.full((L,), t, jnp.int32)].get(
                        mode="promise_in_bounds")
                    for q in range(D_IN // L):
                        rows_v[g * L + t, pl.ds(q * L, L)] = (
                            rows_v[g * L + t, pl.ds(q * L, L)] * wb)
                return carry

            lax.fori_loop(0, n_groups, group, 0)

        SPL = 4
        SK = K // SPL

        def gather_split(src_v, rows_v, gsem):
            for p in range(SPL):
                pltpu.async_copy(
                    feat_hbm.at[src_v.at[pl.ds(p * SK, SK)]],
                    rows_v.at[pl.ds(p * SK, SK)], gsem)

        def drain_split(src_v, rows_v, gsem):
            for p in range(SPL):
                pltpu.make_async_copy(
                    feat_hbm.at[src_v.at[pl.ds(p * SK, SK)]],
                    rows_v.at[pl.ds(p * SK, SK)], gsem).wait()

        def step(i, j, src_v, dst_v, w_v, dsc_v, rows_v, gsem, isem):
            drain_split(src_v, rows_v, gsem)

            mul_block(dst_v, w_v, dsc_v, rows_v, G)

            @pl.when(i < NBF // 2 - 1)
            def _():
                off = base + (j + 2) * K
                pltpu.async_copy(src_hbm.at[pl.ds(off, K)], src_v, isem)
                pltpu.async_copy(dst_hbm.at[pl.ds(off, K)], dst_v, isem)
                pltpu.async_copy(w_hbm.at[pl.ds(off, K)], w_v, isem)

            pltpu.sync_copy(rows_v, acc_sh.at[dsc_v], add=True)

            @pl.when(i < NBF // 2 - 1)
            def _():
                off = base + j * K
                pltpu.make_async_copy(
                    src_hbm.at[pl.ds(off, K)], src_v, isem).wait()
                pltpu.make_async_copy(
                    dst_hbm.at[pl.ds(off, K)], dst_v, isem).wait()
                pltpu.make_async_copy(
                    w_hbm.at[pl.ds(off, K)], w_v, isem).wait()
                gather_split(src_v, rows_v, gsem)

        pltpu.sync_copy(src_hbm.at[pl.ds(base, K)], src_a)
        pltpu.sync_copy(dst_hbm.at[pl.ds(base, K)], dst_a)
        pltpu.sync_copy(w_hbm.at[pl.ds(base, K)], w_a)
        pltpu.sync_copy(src_hbm.at[pl.ds(base + K, K)], src_b)
        pltpu.sync_copy(dst_hbm.at[pl.ds(base + K, K)], dst_b)
        pltpu.sync_copy(w_hbm.at[pl.ds(base + K, K)], w_b)
        gather_split(src_a, rows_a, gsem_a)
        gather_split(src_b, rows_b, gsem_b)

        def pair(i, carry):
            step(i, 2 * i, src_a, dst_a, w_a, dsc_a, rows_a, gsem_a, isem_a)
            step(i, 2 * i + 1, src_b, dst_b, w_b, dsc_b, rows_b, gsem_b,
                 isem_b)
            return carry

        lax.fori_loop(0, NBF // 2, pair, 0)

        toff = base + NBF * K
        pltpu.sync_copy(src_hbm.at[pl.ds(toff, TAIL)], src_t)
        pltpu.sync_copy(dst_hbm.at[pl.ds(toff, TAIL)], dst_t)
        pltpu.sync_copy(w_hbm.at[pl.ds(toff, TAIL)], w_t)
        pltpu.async_copy(feat_hbm.at[src_t],
                         rows_a.at[pl.ds(0, TAIL)], gsem_a).wait()
        plsc.addupdate_scatter(cnt_v, [dst_t[...]], ones)
        wg = w_t[...]
        for t in range(TAIL):
            wb = wg.at[jnp.full((L,), t, jnp.int32)].get(
                mode="promise_in_bounds")
            for q in range(D_IN // L):
                rows_a[t, pl.ds(q * L, L)] = rows_a[t, pl.ds(q * L, L)] * wb
        pltpu.sync_copy(rows_a.at[pl.ds(0, TAIL)], acc_sh.at[dst_t],
                        add=True)

        plsc.subcore_barrier()

        pltpu.sync_copy(acc_sh.at[pl.ds(s * RPT, RPT)],
                        out_hbm.at[c, pl.ds(s * RPT, RPT)])
        pltpu.sync_copy(cnt_v, cnt_hbm.at[c, s])

    return agg(feature, src, dst, w)


def _tc_self(feature, W1, b):
    def body(feat_ref, w_ref, b_ref, out_ref):
        out_ref[...] = jnp.dot(
            feat_ref[...], w_ref[...],
            preferred_element_type=jnp.float32) + b_ref[...]

    return pl.pallas_call(
        body,
        out_shape=jax.ShapeDtypeStruct((N_NODES, D_OUT), jnp.float32),
    )(feature, W1, b.reshape(1, D_OUT))


def _tc_finish(self_out, partials, counts, W2):
    def body(self_ref, part_ref, cnt_ref, w_ref, out_ref):
        p = (part_ref[0] + part_ref[1])[:N_NODES]
        cnt = jnp.sum(cnt_ref[...], axis=(0, 1))[:N_NODES, None]
        mean = p / jnp.maximum(cnt, 1.0)
        out_ref[...] = self_ref[...] + jnp.dot(
            mean, w_ref[...], preferred_element_type=jnp.float32)

    return pl.pallas_call(
        body,
        out_shape=jax.ShapeDtypeStruct((N_NODES, D_OUT), jnp.float32),
    )(self_out, partials, counts, W2)


@jax.jit
def kernel(feature, relation_indices, relation_weight, W, b):
    dst = relation_indices[0].astype(jnp.int32)
    src = relation_indices[1].astype(jnp.int32)
    w = relation_weight.astype(jnp.float32).reshape(-1)
    self_out = _tc_self(feature, W[:D_IN], b)
    partials, counts = _sc_aggregate(feature, src, dst, w)
    return _tc_finish(self_out, partials, counts, W[D_IN:])


# speedup vs baseline: 1.0026x; 1.0026x over previous
import functools

import jax
import jax.numpy as jnp
from jax import lax
from jax.experimental import pallas as pl
from jax.experimental.pallas import tpu as pltpu
from jax.experimental.pallas import tpu_sc as plsc

N_NODES = 10000
N_EDGES = 320000
D_IN = 128
D_OUT = 128

NC = 2
NS = 16
NW = NC * NS
L = 16

K = 128
G = K // L
E_T = N_EDGES // NW
NBF = E_T // K
TAIL = E_T - NBF * K
N_PAD = 10240
RPT = N_PAD // NS


def _sc_aggregate(feature, src, dst, w):
    mesh = plsc.VectorSubcoreMesh(core_axis_name="c", subcore_axis_name="s")

    @functools.partial(
        pl.kernel,
        out_type=(
            jax.ShapeDtypeStruct((NC, N_PAD, D_IN), jnp.float32),
            jax.ShapeDtypeStruct((NC, NS, N_PAD), jnp.float32),
        ),
        mesh=mesh,
        compiler_params=pltpu.CompilerParams(needs_layout_passes=False),
        scratch_types=[
            pltpu.VMEM((K,), jnp.int32),
            pltpu.VMEM((K,), jnp.int32),
            pltpu.VMEM((K,), jnp.int32),
            pltpu.VMEM((K,), jnp.int32),
            pltpu.VMEM((K,), jnp.float32),
            pltpu.VMEM((K,), jnp.float32),
            pltpu.VMEM((K,), jnp.int32),
            pltpu.VMEM((K,), jnp.int32),
            pltpu.VMEM((TAIL,), jnp.int32),
            pltpu.VMEM((TAIL,), jnp.int32),
            pltpu.VMEM((TAIL,), jnp.float32),
            pltpu.VMEM((K, D_IN), jnp.float32),
            pltpu.VMEM((K, D_IN), jnp.float32),
            pltpu.VMEM((N_PAD,), jnp.float32),
            pltpu.VMEM_SHARED((N_PAD, D_IN), jnp.float32),
            pltpu.SemaphoreType.DMA,
            pltpu.SemaphoreType.DMA,
            pltpu.SemaphoreType.DMA,
            pltpu.SemaphoreType.DMA,
        ],
    )
    def agg(feat_hbm, src_hbm, dst_hbm, w_hbm, out_hbm, cnt_hbm,
            src_a, src_b, dst_a, dst_b, w_a, w_b, dsc_a, dsc_b,
            src_t, dst_t, w_t,
            rows_a, rows_b, cnt_v, acc_sh,
            gsem_a, gsem_b, isem_a, isem_b):
        c = lax.axis_index("c")
        s = lax.axis_index("s")
        base = (c * NS + s) * E_T

        zvec = jnp.zeros((L,), jnp.float32)

        def zrow(r, carry):
            for q in range(D_IN // L):
                rows_a[r, pl.ds(q * L, L)] = zvec
            return carry

        lax.fori_loop(0, K, zrow, 0)

        def zcnt(r, carry):
            cnt_v[pl.ds(r * L, L)] = zvec
            return carry

        lax.fori_loop(0, N_PAD // L, zcnt, 0)

        for r in range(RPT // K):
            pltpu.sync_copy(rows_a,
                            acc_sh.at[pl.ds(s * RPT + r * K, K)])

        plsc.subcore_barrier()

        ones = jnp.full((L,), 1.0, jnp.float32)

        def mul_block(dst_v, w_v, dsc_v, rows_v, n_groups):
            def group(g, carry):
                dvec = dst_v[pl.ds(g * L, L)]
                plsc.addupdate_scatter(cnt_v, [dvec], ones)
                dsc_v[pl.ds(g * L, L)] = dvec
                wg = w_v[pl.ds(g * L, L)]
                for t in range(L):
                    wb = wg.at[jnp.full((L,), t, jnp.int32)].get(
                        mode="promise_in_bounds")
                    for q in range(D_IN // L):
                        rows_v[g * L + t, pl.ds(q * L, L)] = (
                            rows_v[g * L + t, pl.ds(q * L, L)] * wb)
                return carry

            lax.fori_loop(0, n_groups, group, 0)

        def step(i, j, src_v, dst_v, w_v, dsc_v, rows_v, gsem, isem):
            pltpu.make_async_copy(feat_hbm.at[src_v], rows_v, gsem).wait()

            mul_block(dst_v, w_v, dsc_v, rows_v, G)

            @pl.when(i < NBF // 2 - 1)
            def _():
                off = base + (j + 2) * K
                pltpu.async_copy(src_hbm.at[pl.ds(off, K)], src_v, isem)
                pltpu.async_copy(dst_hbm.at[pl.ds(off, K)], dst_v, isem)
                pltpu.async_copy(w_hbm.at[pl.ds(off, K)], w_v, isem)

            pltpu.sync_copy(rows_v, acc_sh.at[dsc_v], add=True)

            @pl.when(i < NBF // 2 - 1)
            def _():
                off = base + j * K
                pltpu.make_async_copy(
                    src_hbm.at[pl.ds(off, K)], src_v, isem).wait()
                pltpu.make_async_copy(
                    dst_hbm.at[pl.ds(off, K)], dst_v, isem).wait()
                pltpu.make_async_copy(
                    w_hbm.at[pl.ds(off, K)], w_v, isem).wait()
                pltpu.async_copy(feat_hbm.at[src_v], rows_v, gsem)

        pltpu.sync_copy(src_hbm.at[pl.ds(base, K)], src_a)
        pltpu.sync_copy(dst_hbm.at[pl.ds(base, K)], dst_a)
        pltpu.sync_copy(w_hbm.at[pl.ds(base, K)], w_a)
        pltpu.sync_copy(src_hbm.at[pl.ds(base + K, K)], src_b)
        pltpu.sync_copy(dst_hbm.at[pl.ds(base + K, K)], dst_b)
        pltpu.sync_copy(w_hbm.at[pl.ds(base + K, K)], w_b)
        pltpu.async_copy(feat_hbm.at[src_a], rows_a, gsem_a)
        pltpu.async_copy(feat_hbm.at[src_b], rows_b, gsem_b)

        def pair(i, carry):
            step(i, 2 * i, src_a, dst_a, w_a, dsc_a, rows_a, gsem_a, isem_a)
            step(i, 2 * i + 1, src_b, dst_b, w_b, dsc_b, rows_b, gsem_b,
                 isem_b)
            return carry

        lax.fori_loop(0, NBF // 2, pair, 0)

        toff = base + NBF * K
        pltpu.sync_copy(src_hbm.at[pl.ds(toff, TAIL)], src_t)
        pltpu.sync_copy(dst_hbm.at[pl.ds(toff, TAIL)], dst_t)
        pltpu.sync_copy(w_hbm.at[pl.ds(toff, TAIL)], w_t)
        pltpu.async_copy(feat_hbm.at[src_t],
                         rows_a.at[pl.ds(0, TAIL)], gsem_a).wait()
        plsc.addupdate_scatter(cnt_v, [dst_t[...]], ones)
        wg = w_t[...]
        for t in range(TAIL):
            wb = wg.at[jnp.full((L,), t, jnp.int32)].get(
                mode="promise_in_bounds")
            for q in range(D_IN // L):
                rows_a[t, pl.ds(q * L, L)] = rows_a[t, pl.ds(q * L, L)] * wb
        pltpu.sync_copy(rows_a.at[pl.ds(0, TAIL)], acc_sh.at[dst_t],
                        add=True)

        plsc.subcore_barrier()

        pltpu.sync_copy(acc_sh.at[pl.ds(s * RPT, RPT)],
                        out_hbm.at[c, pl.ds(s * RPT, RPT)])
        pltpu.sync_copy(cnt_v, cnt_hbm.at[c, s])

    return agg(feature, src, dst, w)


def _tc_self(feature, W1, b):
    def body(feat_ref, w_ref, b_ref, out_ref):
        out_ref[...] = jnp.dot(
            feat_ref[...], w_ref[...],
            preferred_element_type=jnp.float32) + b_ref[...]

    return pl.pallas_call(
        body,
        out_shape=jax.ShapeDtypeStruct((N_NODES, D_OUT), jnp.float32),
    )(feature, W1, b.reshape(1, D_OUT))


def _tc_finish(self_out, partials, counts, W2):
    def body(self_ref, part_ref, cnt_ref, w_ref, out_ref):
        p = (part_ref[0] + part_ref[1])[:N_NODES]
        cnt = jnp.sum(cnt_ref[...], axis=(0, 1))[:N_NODES, None]
        mean = p / jnp.maximum(cnt, 1.0)
        out_ref[...] = self_ref[...] + jnp.dot(
            mean, w_ref[...], preferred_element_type=jnp.float32)

    return pl.pallas_call(
        body,
        out_shape=jax.ShapeDtypeStruct((N_NODES, D_OUT), jnp.float32),
    )(self_out, partials, counts, W2)


@jax.jit
def kernel(feature, relation_indices, relation_weight, W, b):
    dst = relation_indices[0].astype(jnp.int32)
    src = relation_indices[1].astype(jnp.int32)
    w = relation_weight.astype(jnp.float32).reshape(-1)
    self_out = _tc_self(feature, W[:D_IN], b)
    partials, counts = _sc_aggregate(feature, src, dst, w)
    return _tc_finish(self_out, partials, counts, W[D_IN:])
